# Initial kernel scaffold; baseline (speedup 1.0000x reference)
import functools

import jax
import jax.numpy as jnp
from jax import lax
from jax.experimental import pallas as pl
from jax.experimental.pallas import tpu as pltpu
from jax.experimental.pallas import tpu_sc as plsc

N = 10000
E = 320000
C = 128
EPS_MIX = 0.1

NC = 2
NS = 16
NW = NC * NS
K = 80
NB = E // (NW * K)
RPT = N // NS
ZR = 125


def _gate_kernel(x_ref, w_ref, b_ref, o_ref):
    o_ref[...] = (
        jnp.dot(x_ref[...], w_ref[...], preferred_element_type=jnp.float32)
        + b_ref[...]
    )


def _combine_kernel(p_ref, d_ref, x_ref, o_ref):
    p = p_ref[0] + p_ref[1]
    den = d_ref[0, :, 0:1] + d_ref[1, :, 0:1]
    o_ref[...] = (1.0 - EPS_MIX) * p / (den + 1e-16) + EPS_MIX * x_ref[...]


def _sc_edge_kernel(row_hbm, col_hbm, srow_hbm, scol_hbm, x_hbm,
                    prop_out, den_out,
                    rowv, colv, srow, scol, exb, xg, sbuf, dbuf, zb, zdb,
                    propS, denS, sem):
    c = lax.axis_index("c")
    s = lax.axis_index("s")
    w = c * NS + s

    pltpu.sync_copy(row_hbm.at[pl.ds(w * NB, NB)], rowv)
    pltpu.sync_copy(col_hbm.at[pl.ds(w * NB, NB)], colv)
    pltpu.sync_copy(srow_hbm, srow)
    pltpu.sync_copy(scol_hbm, scol)

    zeros16 = jnp.zeros((16,), jnp.float32)

    def zfill(i, _):
        for cc in range(C // 16):
            zb[i, pl.ds(cc * 16, 16)] = zeros16
        zdb[i, pl.ds(0, 16)] = zeros16
        return ()

    lax.fori_loop(0, ZR, zfill, ())

    def zcopy(j, _):
        pltpu.sync_copy(zb, propS.at[pl.ds(s * RPT + j * ZR, ZR)])
        pltpu.sync_copy(zdb, denS.at[pl.ds(s * RPT + j * ZR, ZR)])
        return ()

    lax.fori_loop(0, RPT // ZR, zcopy, ())
    plsc.subcore_barrier()

    lane = lax.broadcasted_iota(jnp.int32, (16,), 0)

    def batch(b, _):
        pltpu.sync_copy(x_hbm.at[rowv.at[b]], xg)

        for j in range(K // 16):
            ri = rowv[b, pl.ds(j * 16, 16)]
            ci = colv[b, pl.ds(j * 16, 16)]
            a = plsc.load_gather(srow, [ri])
            bb = plsc.load_gather(scol, [ci])
            z = a + bb
            t = 1.0 - 2.0 / (1.0 + jnp.exp(2.0 * z))
            exb[pl.ds(j * 16, 16)] = jnp.exp(t)

        def rowloop(i, _):
            e = exb[i]
            for cc in range(C // 16):
                sbuf[i, pl.ds(cc * 16, 16)] = xg[i, pl.ds(cc * 16, 16)] * e
            dbuf[i, pl.ds(0, 16)] = jnp.where(lane == 0, e, 0.0)
            return ()

        lax.fori_loop(0, K, rowloop, ())

        pltpu.sync_copy(sbuf, propS.at[colv.at[b]], add=True)
        pltpu.sync_copy(dbuf, denS.at[colv.at[b]], add=True)
        return ()

    lax.fori_loop(0, NB, batch, ())

    plsc.subcore_barrier()
    pltpu.sync_copy(propS.at[pl.ds(s * RPT, RPT)],
                    prop_out.at[c, pl.ds(s * RPT, RPT)])
    pltpu.sync_copy(denS.at[pl.ds(s * RPT, RPT)],
                    den_out.at[c, pl.ds(s * RPT, RPT)])


_sc_edge = functools.partial(
    pl.kernel,
    out_type=(
        jax.ShapeDtypeStruct((NC, N, C), jnp.float32),
        jax.ShapeDtypeStruct((NC, N, 16), jnp.float32),
    ),
    mesh=plsc.VectorSubcoreMesh(core_axis_name="c", subcore_axis_name="s"),
    scratch_types=[
        pltpu.VMEM((NB, K), jnp.int32),
        pltpu.VMEM((NB, K), jnp.int32),
        pltpu.VMEM((N,), jnp.float32),
        pltpu.VMEM((N,), jnp.float32),
        pltpu.VMEM((K,), jnp.float32),
        pltpu.VMEM((K, C), jnp.float32),
        pltpu.VMEM((K, C), jnp.float32),
        pltpu.VMEM((K, 16), jnp.float32),
        pltpu.VMEM((ZR, C), jnp.float32),
        pltpu.VMEM((ZR, 16), jnp.float32),
        pltpu.MemorySpace.VMEM_SHARED((N, C), jnp.float32),
        pltpu.MemorySpace.VMEM_SHARED((N, 16), jnp.float32),
        pltpu.SemaphoreType.DMA,
    ],
)(_sc_edge_kernel)


@jax.jit
def kernel(x, edge_index, gate_w, gate_b):
    row = edge_index[0].astype(jnp.int32).reshape(E // K, K)
    col = edge_index[1].astype(jnp.int32).reshape(E // K, K)

    wcat = jnp.concatenate(
        [gate_w[0, :C, None], gate_w[0, C:, None]], axis=1)
    bias = jnp.stack([jnp.zeros((), jnp.float32), gate_b[0]])[None, :]
    BN = 2000
    s2 = pl.pallas_call(
        _gate_kernel,
        out_shape=jax.ShapeDtypeStruct((N, 2), jnp.float32),
        grid=(N // BN,),
        in_specs=[
            pl.BlockSpec((BN, C), lambda i: (i, 0)),
            pl.BlockSpec((C, 2), lambda i: (0, 0)),
            pl.BlockSpec((1, 2), lambda i: (0, 0)),
        ],
        out_specs=pl.BlockSpec((BN, 2), lambda i: (i, 0)),
    )(x, wcat, bias)
    s_row = s2[:, 0]
    s_col = s2[:, 1]

    prop, den = _sc_edge(row, col, s_row, s_col, x)

    R = 1000
    out = pl.pallas_call(
        _combine_kernel,
        out_shape=jax.ShapeDtypeStruct((N, C), jnp.float32),
        grid=(N // R,),
        in_specs=[
            pl.BlockSpec((NC, R, C), lambda i: (0, i, 0)),
            pl.BlockSpec((NC, R, 16), lambda i: (0, i, 0)),
            pl.BlockSpec((R, C), lambda i: (i, 0)),
        ],
        out_specs=pl.BlockSpec((R, C), lambda i: (i, 0)),
    )(prop, den, x)
    return out


# speedup vs baseline: 6.7834x; 6.7834x over previous
import functools

import jax
import jax.numpy as jnp
from jax import lax
from jax.experimental import pallas as pl
from jax.experimental.pallas import tpu as pltpu
from jax.experimental.pallas import tpu_sc as plsc

N = 10000
E = 320000
C = 128
EPS_MIX = 0.1

NC = 2
NS = 16
NW = NC * NS
EPT = E // NW
K = 128
NBP = 80
EPP = NBP * K
SB = 8
NSB = NBP // SB
G16 = EPT // 16
STRIPE = 624
LAST = N - STRIPE * (NS - 1)
ZR = 8


def _gate_kernel(x_ref, w_ref, b_ref, o_ref):
    o_ref[...] = (
        jnp.dot(x_ref[...], w_ref[...], preferred_element_type=jnp.float32)
        + b_ref[...]
    )


def _combine_kernel(p_ref, d_ref, x_ref, o_ref):
    den = d_ref[0, :, 0:1]
    half = C // NC
    o_ref[:, :half] = ((1.0 - EPS_MIX) * p_ref[0] / (den + 1e-16)
                       + EPS_MIX * x_ref[:, :half])
    o_ref[:, half:] = ((1.0 - EPS_MIX) * p_ref[1] / (den + 1e-16)
                       + EPS_MIX * x_ref[:, half:])


def _sc_ex_body(row_hbm, col_hbm, srow_hbm, scol_hbm, ex_hbm,
                rowv, colv, srow, scol, exv):
    c = lax.axis_index("c")
    s = lax.axis_index("s")
    w = c * NS + s

    pltpu.sync_copy(row_hbm.at[w], rowv)
    pltpu.sync_copy(col_hbm.at[w], colv)
    pltpu.sync_copy(srow_hbm, srow)
    pltpu.sync_copy(scol_hbm, scol)

    def group(r, _):
        for j in range(K // 16):
            ri = rowv[r, pl.ds(j * 16, 16)]
            ci = colv[r, pl.ds(j * 16, 16)]
            a = plsc.load_gather(srow, [ri])
            b = plsc.load_gather(scol, [ci])
            z = a + b
            t = 1.0 - 2.0 / (1.0 + jnp.exp(2.0 * z))
            ex = jnp.exp(t)
            gid = jnp.full((16,), r * (K // 16) + j, jnp.int32)
            exv[r, pl.ds(j * 16, 16)] = jnp.where(gid < G16, ex, 0.0)
        return ()

    lax.fori_loop(0, NBP, group, ())
    pltpu.sync_copy(exv, ex_hbm.at[w])


_sc_ex = functools.partial(
    pl.kernel,
    out_type=jax.ShapeDtypeStruct((NW, NBP, K), jnp.float32),
    mesh=plsc.VectorSubcoreMesh(core_axis_name="c", subcore_axis_name="s"),
    compiler_params=pltpu.CompilerParams(needs_layout_passes=False),
    scratch_types=[
        pltpu.VMEM((NBP, K), jnp.int32),
        pltpu.VMEM((NBP, K), jnp.int32),
        pltpu.VMEM((N,), jnp.float32),
        pltpu.VMEM((N,), jnp.float32),
        pltpu.VMEM((NBP, K), jnp.float32),
    ],
)(_sc_ex_body)


CH = C // NC
NB2 = 160
NSB2 = NB2 // SB


def _sc_heavy_body(row_hbm, col_hbm, ex_hbm, xf_hbm,
                   prop_out, den_out,
                   rowc, colc, exc, idxc, xg, dbuf, zb, zdb,
                   propS, denS, sem, sem2):
    c = lax.axis_index("c")
    s = lax.axis_index("s")

    zeros16 = jnp.zeros((16,), jnp.float32)

    def zfill(i, _):
        for cc in range(CH // 16):
            zb[i, pl.ds(cc * 16, 16)] = zeros16
        zdb[i, pl.ds(0, 16)] = zeros16
        return ()

    lax.fori_loop(0, ZR, zfill, ())

    def zcopy(j, _):
        o = s * STRIPE + j * ZR
        pltpu.async_copy(zb, propS.at[pl.ds(o, ZR)], sem).wait()
        pltpu.async_copy(zdb, denS.at[pl.ds(o, ZR)], sem2).wait()
        return ()

    lax.fori_loop(0, STRIPE // ZR, zcopy, ())

    @pl.when(s == NS - 1)
    def _():
        for j in range((LAST - STRIPE) // ZR):
            o = N - LAST + STRIPE + j * ZR
            pltpu.async_copy(zb, propS.at[pl.ds(o, ZR)], sem).wait()
            pltpu.async_copy(zdb, denS.at[pl.ds(o, ZR)], sem2).wait()

    plsc.subcore_barrier()

    lane = lax.broadcasted_iota(jnp.int32, (16,), 0)

    def super_batch(sb, _):
        o = pl.multiple_of(sb * SB, SB)
        pltpu.sync_copy(row_hbm.at[s, pl.ds(o, SB)], rowc)
        pltpu.sync_copy(col_hbm.at[s, pl.ds(o, SB)], colc)
        pltpu.sync_copy(ex_hbm.at[s, pl.ds(o, SB)], exc)

        def adj(jr, _):
            for g in range(K // 16):
                idxc[jr, pl.ds(g * 16, 16)] = (
                    rowc[jr, pl.ds(g * 16, 16)] * 2 + c)
            return ()

        lax.fori_loop(0, SB, adj, ())

        for j in range(SB):
            pltpu.sync_copy(xf_hbm.at[idxc.at[j]], xg)

            def rowloop(i, _, j=j):
                e = plsc.load_gather(
                    exc, [jnp.full((16,), j, jnp.int32),
                          jnp.full((16,), i, jnp.int32)])
                for cc in range(CH // 16):
                    xg[i, pl.ds(cc * 16, 16)] = xg[i, pl.ds(cc * 16, 16)] * e
                dbuf[i, pl.ds(0, 16)] = jnp.where(lane == 0, e, 0.0)
                return ()

            lax.fori_loop(0, K, rowloop, ())

            pltpu.sync_copy(xg, propS.at[colc.at[j]], add=True)
            pltpu.sync_copy(dbuf, denS.at[colc.at[j]], add=True)
        return ()

    lax.fori_loop(0, NSB2, super_batch, ())

    plsc.subcore_barrier()

    def ocopy(j, _):
        o = s * STRIPE + j * ZR
        pltpu.async_copy(propS.at[pl.ds(o, ZR)],
                         prop_out.at[c, pl.ds(o, ZR)], sem).wait()
        pltpu.async_copy(denS.at[pl.ds(o, ZR)],
                         den_out.at[c, pl.ds(o, ZR)], sem2).wait()
        return ()

    lax.fori_loop(0, STRIPE // ZR, ocopy, ())

    @pl.when(s == NS - 1)
    def _():
        for j in range((LAST - STRIPE) // ZR):
            o = N - LAST + STRIPE + j * ZR
            pltpu.async_copy(propS.at[pl.ds(o, ZR)],
                             prop_out.at[c, pl.ds(o, ZR)], sem).wait()
            pltpu.async_copy(denS.at[pl.ds(o, ZR)],
                             den_out.at[c, pl.ds(o, ZR)], sem2).wait()


_sc_heavy = functools.partial(
    pl.kernel,
    out_type=(
        jax.ShapeDtypeStruct((NC, N, CH), jnp.float32),
        jax.ShapeDtypeStruct((NC, N, 16), jnp.float32),
    ),
    mesh=plsc.VectorSubcoreMesh(core_axis_name="c", subcore_axis_name="s"),
    compiler_params=pltpu.CompilerParams(
        needs_layout_passes=False, use_tc_tiling_on_sc=False),
    scratch_types=[
        pltpu.VMEM((SB, K), jnp.int32),
        pltpu.VMEM((SB, K), jnp.int32),
        pltpu.VMEM((SB, K), jnp.float32),
        pltpu.VMEM((SB, K), jnp.int32),
        pltpu.VMEM((K, CH), jnp.float32),
        pltpu.VMEM((K, 16), jnp.float32),
        pltpu.VMEM((ZR, CH), jnp.float32),
        pltpu.VMEM((ZR, 16), jnp.float32),
        pltpu.MemorySpace.VMEM_SHARED((N, CH), jnp.float32),
        pltpu.MemorySpace.VMEM_SHARED((N, 16), jnp.float32),
        pltpu.SemaphoreType.DMA,
        pltpu.SemaphoreType.DMA,
    ],
)(_sc_heavy_body)


@jax.jit
def kernel(x, edge_index, gate_w, gate_b):
    pad = NW * EPP - E
    row = jnp.pad(edge_index[0].astype(jnp.int32).reshape(NW, EPT),
                  ((0, 0), (0, EPP - EPT))).reshape(NW, NBP, K)
    col = jnp.pad(edge_index[1].astype(jnp.int32).reshape(NW, EPT),
                  ((0, 0), (0, EPP - EPT))).reshape(NW, NBP, K)
    del pad

    wcat = jnp.concatenate(
        [gate_w[0, :C, None], gate_w[0, C:, None]], axis=1)
    bias = jnp.stack([jnp.zeros((), jnp.float32), gate_b[0]])[None, :]
    BN = 2000
    s2 = pl.pallas_call(
        _gate_kernel,
        out_shape=jax.ShapeDtypeStruct((N, 2), jnp.float32),
        grid=(N // BN,),
        in_specs=[
            pl.BlockSpec((BN, C), lambda i: (i, 0)),
            pl.BlockSpec((C, 2), lambda i: (0, 0)),
            pl.BlockSpec((1, 2), lambda i: (0, 0)),
        ],
        out_specs=pl.BlockSpec((BN, 2), lambda i: (i, 0)),
    )(x, wcat, bias)
    s_row = s2[:, 0]
    s_col = s2[:, 1]

    ex = _sc_ex(row, col, s_row, s_col)

    row16 = row.reshape(NS, NB2, K)
    col16 = col.reshape(NS, NB2, K)
    ex16 = ex.reshape(NS, NB2, K)
    xf = x.reshape(2 * N, CH)
    prop, den = _sc_heavy(row16, col16, ex16, xf)

    R = 1000
    out = pl.pallas_call(
        _combine_kernel,
        out_shape=jax.ShapeDtypeStruct((N, C), jnp.float32),
        grid=(N // R,),
        in_specs=[
            pl.BlockSpec((NC, R, C // NC), lambda i: (0, i, 0)),
            pl.BlockSpec((NC, R, 16), lambda i: (0, i, 0)),
            pl.BlockSpec((R, C), lambda i: (i, 0)),
        ],
        out_specs=pl.BlockSpec((R, C), lambda i: (i, 0)),
    )(prop, den, x)
    return out


# speedup vs baseline: 7.7938x; 1.1490x over previous
import functools

import jax
import jax.numpy as jnp
from jax import lax
from jax.experimental import pallas as pl
from jax.experimental.pallas import tpu as pltpu
from jax.experimental.pallas import tpu_sc as plsc

N = 10000
E = 320000
C = 128
EPS_MIX = 0.1

NC = 2
NS = 16
NW = NC * NS
EPT = E // NW
K = 128
NBP = 80
EPP = NBP * K
SB = 8
NSB = NBP // SB
G16 = EPT // 16
STRIPE = 624
LAST = N - STRIPE * (NS - 1)
ZR = 8


def _gate_kernel(x_ref, w_ref, b_ref, o_ref):
    o_ref[...] = (
        jnp.dot(x_ref[...], w_ref[...], preferred_element_type=jnp.float32)
        + b_ref[...]
    )


def _combine_kernel(p_ref, d_ref, x_ref, o_ref):
    den = d_ref[0, :, 0:1]
    half = C // NC
    o_ref[:, :half] = ((1.0 - EPS_MIX) * p_ref[0] / (den + 1e-16)
                       + EPS_MIX * x_ref[:, :half])
    o_ref[:, half:] = ((1.0 - EPS_MIX) * p_ref[1] / (den + 1e-16)
                       + EPS_MIX * x_ref[:, half:])


def _sc_ex_body(row_hbm, col_hbm, srow_hbm, scol_hbm, ex_hbm,
                rowv, colv, srow, scol, exv):
    c = lax.axis_index("c")
    s = lax.axis_index("s")
    w = c * NS + s

    pltpu.sync_copy(row_hbm.at[w], rowv)
    pltpu.sync_copy(col_hbm.at[w], colv)
    pltpu.sync_copy(srow_hbm, srow)
    pltpu.sync_copy(scol_hbm, scol)

    def group(r, _):
        for j in range(K // 16):
            ri = rowv[r, pl.ds(j * 16, 16)]
            ci = colv[r, pl.ds(j * 16, 16)]
            a = plsc.load_gather(srow, [ri])
            b = plsc.load_gather(scol, [ci])
            z = a + b
            t = 1.0 - 2.0 / (1.0 + jnp.exp(2.0 * z))
            ex = jnp.exp(t)
            gid = jnp.full((16,), r * (K // 16) + j, jnp.int32)
            exv[r, pl.ds(j * 16, 16)] = jnp.where(gid < G16, ex, 0.0)
        return ()

    lax.fori_loop(0, NBP, group, ())
    pltpu.sync_copy(exv, ex_hbm.at[w])


_sc_ex = functools.partial(
    pl.kernel,
    out_type=jax.ShapeDtypeStruct((NW, NBP, K), jnp.float32),
    mesh=plsc.VectorSubcoreMesh(core_axis_name="c", subcore_axis_name="s"),
    compiler_params=pltpu.CompilerParams(needs_layout_passes=False),
    scratch_types=[
        pltpu.VMEM((NBP, K), jnp.int32),
        pltpu.VMEM((NBP, K), jnp.int32),
        pltpu.VMEM((N,), jnp.float32),
        pltpu.VMEM((N,), jnp.float32),
        pltpu.VMEM((NBP, K), jnp.float32),
    ],
)(_sc_ex_body)


CH = C // NC
NB2 = 160
NSB2 = NB2 // SB


def _sc_heavy_body(row_hbm, col_hbm, ex_hbm, xf_hbm,
                   prop_out, den_out,
                   rowc, colc, exc, idxc, xg, xg2, dbuf, zb, zdb,
                   propS, denS, sem, sem2, sem3):
    c = lax.axis_index("c")
    s = lax.axis_index("s")

    zeros16 = jnp.zeros((16,), jnp.float32)

    def zfill(i, _):
        for cc in range(CH // 16):
            zb[i, pl.ds(cc * 16, 16)] = zeros16
        zdb[i, pl.ds(0, 16)] = zeros16
        return ()

    lax.fori_loop(0, ZR, zfill, ())

    def zcopy(j, _):
        o = s * STRIPE + j * ZR
        pltpu.async_copy(zb, propS.at[pl.ds(o, ZR)], sem).wait()
        pltpu.async_copy(zdb, denS.at[pl.ds(o, ZR)], sem2).wait()
        return ()

    lax.fori_loop(0, STRIPE // ZR, zcopy, ())

    @pl.when(s == NS - 1)
    def _():
        for j in range((LAST - STRIPE) // ZR):
            o = N - LAST + STRIPE + j * ZR
            pltpu.async_copy(zb, propS.at[pl.ds(o, ZR)], sem).wait()
            pltpu.async_copy(zdb, denS.at[pl.ds(o, ZR)], sem2).wait()

    plsc.subcore_barrier()

    lane = lax.broadcasted_iota(jnp.int32, (16,), 0)

    def super_batch(sb, _):
        o = pl.multiple_of(sb * SB, SB)
        pltpu.sync_copy(row_hbm.at[s, pl.ds(o, SB)], rowc)
        pltpu.sync_copy(col_hbm.at[s, pl.ds(o, SB)], colc)
        pltpu.sync_copy(ex_hbm.at[s, pl.ds(o, SB)], exc)

        def adj(jr, _):
            for g in range(K // 16):
                idxc[jr, pl.ds(g * 16, 16)] = (
                    rowc[jr, pl.ds(g * 16, 16)] * 2 + c)
            return ()

        lax.fori_loop(0, SB, adj, ())

        bufs = (xg, xg2)
        g = pltpu.async_copy(xf_hbm.at[idxc.at[0]], bufs[0], sem3)
        for j in range(SB):
            g.wait()
            if j + 1 < SB:
                g = pltpu.async_copy(
                    xf_hbm.at[idxc.at[j + 1]], bufs[(j + 1) % 2], sem3)
            buf = bufs[j % 2]

            def rowloop(i, _, j=j, buf=buf):
                e = plsc.load_gather(
                    exc, [jnp.full((16,), j, jnp.int32),
                          jnp.full((16,), i, jnp.int32)])
                for cc in range(CH // 16):
                    buf[i, pl.ds(cc * 16, 16)] = buf[i, pl.ds(cc * 16, 16)] * e
                dbuf[i, pl.ds(0, 16)] = jnp.where(lane == 0, e, 0.0)
                return ()

            lax.fori_loop(0, K, rowloop, ())

            pltpu.sync_copy(buf, propS.at[colc.at[j]], add=True)
            pltpu.sync_copy(dbuf, denS.at[colc.at[j]], add=True)
        return ()

    lax.fori_loop(0, NSB2, super_batch, ())

    plsc.subcore_barrier()

    def ocopy(j, _):
        o = s * STRIPE + j * ZR
        pltpu.async_copy(propS.at[pl.ds(o, ZR)],
                         prop_out.at[c, pl.ds(o, ZR)], sem).wait()
        pltpu.async_copy(denS.at[pl.ds(o, ZR)],
                         den_out.at[c, pl.ds(o, ZR)], sem2).wait()
        return ()

    lax.fori_loop(0, STRIPE // ZR, ocopy, ())

    @pl.when(s == NS - 1)
    def _():
        for j in range((LAST - STRIPE) // ZR):
            o = N - LAST + STRIPE + j * ZR
            pltpu.async_copy(propS.at[pl.ds(o, ZR)],
                             prop_out.at[c, pl.ds(o, ZR)], sem).wait()
            pltpu.async_copy(denS.at[pl.ds(o, ZR)],
                             den_out.at[c, pl.ds(o, ZR)], sem2).wait()


_sc_heavy = functools.partial(
    pl.kernel,
    out_type=(
        jax.ShapeDtypeStruct((NC, N, CH), jnp.float32),
        jax.ShapeDtypeStruct((NC, N, 16), jnp.float32),
    ),
    mesh=plsc.VectorSubcoreMesh(core_axis_name="c", subcore_axis_name="s"),
    compiler_params=pltpu.CompilerParams(
        needs_layout_passes=False, use_tc_tiling_on_sc=False),
    scratch_types=[
        pltpu.VMEM((SB, K), jnp.int32),
        pltpu.VMEM((SB, K), jnp.int32),
        pltpu.VMEM((SB, K), jnp.float32),
        pltpu.VMEM((SB, K), jnp.int32),
        pltpu.VMEM((K, CH), jnp.float32),
        pltpu.VMEM((K, CH), jnp.float32),
        pltpu.VMEM((K, 16), jnp.float32),
        pltpu.VMEM((ZR, CH), jnp.float32),
        pltpu.VMEM((ZR, 16), jnp.float32),
        pltpu.MemorySpace.VMEM_SHARED((N, CH), jnp.float32),
        pltpu.MemorySpace.VMEM_SHARED((N, 16), jnp.float32),
        pltpu.SemaphoreType.DMA,
        pltpu.SemaphoreType.DMA,
        pltpu.SemaphoreType.DMA,
    ],
)(_sc_heavy_body)


@jax.jit
def kernel(x, edge_index, gate_w, gate_b):
    pad = NW * EPP - E
    row = jnp.pad(edge_index[0].astype(jnp.int32).reshape(NW, EPT),
                  ((0, 0), (0, EPP - EPT))).reshape(NW, NBP, K)
    col = jnp.pad(edge_index[1].astype(jnp.int32).reshape(NW, EPT),
                  ((0, 0), (0, EPP - EPT))).reshape(NW, NBP, K)
    del pad

    wcat = jnp.concatenate(
        [gate_w[0, :C, None], gate_w[0, C:, None]], axis=1)
    bias = jnp.stack([jnp.zeros((), jnp.float32), gate_b[0]])[None, :]
    BN = 2000
    s2 = pl.pallas_call(
        _gate_kernel,
        out_shape=jax.ShapeDtypeStruct((N, 2), jnp.float32),
        grid=(N // BN,),
        in_specs=[
            pl.BlockSpec((BN, C), lambda i: (i, 0)),
            pl.BlockSpec((C, 2), lambda i: (0, 0)),
            pl.BlockSpec((1, 2), lambda i: (0, 0)),
        ],
        out_specs=pl.BlockSpec((BN, 2), lambda i: (i, 0)),
    )(x, wcat, bias)
    s_row = s2[:, 0]
    s_col = s2[:, 1]

    ex = _sc_ex(row, col, s_row, s_col)

    row16 = row.reshape(NS, NB2, K)
    col16 = col.reshape(NS, NB2, K)
    ex16 = ex.reshape(NS, NB2, K)
    xf = x.reshape(2 * N, CH)
    prop, den = _sc_heavy(row16, col16, ex16, xf)

    R = 1000
    out = pl.pallas_call(
        _combine_kernel,
        out_shape=jax.ShapeDtypeStruct((N, C), jnp.float32),
        grid=(N // R,),
        in_specs=[
            pl.BlockSpec((NC, R, C // NC), lambda i: (0, i, 0)),
            pl.BlockSpec((NC, R, 16), lambda i: (0, i, 0)),
            pl.BlockSpec((R, C), lambda i: (i, 0)),
        ],
        out_specs=pl.BlockSpec((R, C), lambda i: (i, 0)),
    )(prop, den, x)
    return out


# speedup vs baseline: 7.9134x; 1.0153x over previous
import functools

import jax
import jax.numpy as jnp
from jax import lax
from jax.experimental import pallas as pl
from jax.experimental.pallas import tpu as pltpu
from jax.experimental.pallas import tpu_sc as plsc

N = 10000
E = 320000
C = 128
EPS_MIX = 0.1

NC = 2
NS = 16
NW = NC * NS
EPT = E // NW
K = 128
NBP = 80
EPP = NBP * K
SB = 8
NSB = NBP // SB
G16 = EPT // 16
STRIPE = 624
LAST = N - STRIPE * (NS - 1)
ZR = 8


def _gate_kernel(x_ref, w_ref, b_ref, o_ref):
    o_ref[...] = (
        jnp.dot(x_ref[...], w_ref[...], preferred_element_type=jnp.float32)
        + b_ref[...]
    )


def _combine_kernel(p_ref, d_ref, x_ref, o_ref):
    den = d_ref[0, :, 0:1]
    half = C // NC
    o_ref[:, :half] = ((1.0 - EPS_MIX) * p_ref[0] / (den + 1e-16)
                       + EPS_MIX * x_ref[:, :half])
    o_ref[:, half:] = ((1.0 - EPS_MIX) * p_ref[1] / (den + 1e-16)
                       + EPS_MIX * x_ref[:, half:])


def _sc_ex_body(row_hbm, col_hbm, srow_hbm, scol_hbm, ex_hbm,
                rowv, colv, srow, scol, exv):
    c = lax.axis_index("c")
    s = lax.axis_index("s")
    w = c * NS + s

    pltpu.sync_copy(row_hbm.at[w], rowv)
    pltpu.sync_copy(col_hbm.at[w], colv)
    pltpu.sync_copy(srow_hbm, srow)
    pltpu.sync_copy(scol_hbm, scol)

    def group(r, _):
        for j in range(K // 16):
            ri = rowv[r, pl.ds(j * 16, 16)]
            ci = colv[r, pl.ds(j * 16, 16)]
            a = plsc.load_gather(srow, [ri])
            b = plsc.load_gather(scol, [ci])
            z = a + b
            t = 1.0 - 2.0 / (1.0 + jnp.exp(2.0 * z))
            ex = jnp.exp(t)
            gid = jnp.full((16,), r * (K // 16) + j, jnp.int32)
            exv[r, pl.ds(j * 16, 16)] = jnp.where(gid < G16, ex, 0.0)
        return ()

    lax.fori_loop(0, NBP, group, ())
    pltpu.sync_copy(exv, ex_hbm.at[w])


_sc_ex = functools.partial(
    pl.kernel,
    out_type=jax.ShapeDtypeStruct((NW, NBP, K), jnp.float32),
    mesh=plsc.VectorSubcoreMesh(core_axis_name="c", subcore_axis_name="s"),
    compiler_params=pltpu.CompilerParams(needs_layout_passes=False),
    scratch_types=[
        pltpu.VMEM((NBP, K), jnp.int32),
        pltpu.VMEM((NBP, K), jnp.int32),
        pltpu.VMEM((N,), jnp.float32),
        pltpu.VMEM((N,), jnp.float32),
        pltpu.VMEM((NBP, K), jnp.float32),
    ],
)(_sc_ex_body)


CH = C // NC
NB2 = 160
NSB2 = NB2 // SB


def _sc_heavy_body(row_hbm, col_hbm, ex_hbm, xf_hbm,
                   prop_out, den_out,
                   rowc, colc, exc, idxc, xg, xg2, dbuf, zb, zdb,
                   propS, denS, sem, sem2, sem3, sem4):
    c = lax.axis_index("c")
    s = lax.axis_index("s")

    zeros16 = jnp.zeros((16,), jnp.float32)

    def zfill(i, _):
        for cc in range(CH // 16):
            zb[i, pl.ds(cc * 16, 16)] = zeros16
        zdb[i, pl.ds(0, 16)] = zeros16
        return ()

    lax.fori_loop(0, ZR, zfill, ())

    def zcopy(j, _):
        o = s * STRIPE + j * ZR
        pltpu.async_copy(zb, propS.at[pl.ds(o, ZR)], sem).wait()
        pltpu.async_copy(zdb, denS.at[pl.ds(o, ZR)], sem2).wait()
        return ()

    lax.fori_loop(0, STRIPE // ZR, zcopy, ())

    @pl.when(s == NS - 1)
    def _():
        for j in range((LAST - STRIPE) // ZR):
            o = N - LAST + STRIPE + j * ZR
            pltpu.async_copy(zb, propS.at[pl.ds(o, ZR)], sem).wait()
            pltpu.async_copy(zdb, denS.at[pl.ds(o, ZR)], sem2).wait()

    plsc.subcore_barrier()

    lane = lax.broadcasted_iota(jnp.int32, (16,), 0)

    def super_batch(sb, _):
        o = pl.multiple_of(sb * SB, SB)
        pltpu.sync_copy(row_hbm.at[s, pl.ds(o, SB)], rowc)
        pltpu.sync_copy(col_hbm.at[s, pl.ds(o, SB)], colc)
        pltpu.sync_copy(ex_hbm.at[s, pl.ds(o, SB)], exc)

        def adj(jr, _):
            for g in range(K // 16):
                idxc[jr, pl.ds(g * 16, 16)] = (
                    rowc[jr, pl.ds(g * 16, 16)] * 2 + c)
            return ()

        lax.fori_loop(0, SB, adj, ())

        bufs = (xg, xg2)
        sc_sems = (sem, sem2)
        pend = [None, None]
        pend_d = [None]
        g = pltpu.async_copy(xf_hbm.at[idxc.at[0]], bufs[0], sem3)
        for j in range(SB):
            g.wait()
            if j + 1 < SB:
                nxt = (j + 1) % 2
                if pend[nxt] is not None:
                    pend[nxt].wait()
                    pend[nxt] = None
                g = pltpu.async_copy(xf_hbm.at[idxc.at[j + 1]], bufs[nxt],
                                     sem3)
            buf = bufs[j % 2]
            if pend_d[0] is not None:
                pend_d[0].wait()
                pend_d[0] = None

            def rowloop(i, _, j=j, buf=buf):
                e = plsc.load_gather(
                    exc, [jnp.full((16,), j, jnp.int32),
                          jnp.full((16,), i, jnp.int32)])
                for cc in range(CH // 16):
                    buf[i, pl.ds(cc * 16, 16)] = buf[i, pl.ds(cc * 16, 16)] * e
                dbuf[i, pl.ds(0, 16)] = jnp.where(lane == 0, e, 0.0)
                return ()

            lax.fori_loop(0, K, rowloop, ())

            pend[j % 2] = pltpu.async_copy(
                buf, propS.at[colc.at[j]], sc_sems[j % 2], add=True)
            pend_d[0] = pltpu.async_copy(
                dbuf, denS.at[colc.at[j]], sem4, add=True)
        for p in pend:
            if p is not None:
                p.wait()
        pend_d[0].wait()
        return ()

    lax.fori_loop(0, NSB2, super_batch, ())

    plsc.subcore_barrier()

    def ocopy(j, _):
        o = s * STRIPE + j * ZR
        pltpu.async_copy(propS.at[pl.ds(o, ZR)],
                         prop_out.at[c, pl.ds(o, ZR)], sem).wait()
        pltpu.async_copy(denS.at[pl.ds(o, ZR)],
                         den_out.at[c, pl.ds(o, ZR)], sem2).wait()
        return ()

    lax.fori_loop(0, STRIPE // ZR, ocopy, ())

    @pl.when(s == NS - 1)
    def _():
        for j in range((LAST - STRIPE) // ZR):
            o = N - LAST + STRIPE + j * ZR
            pltpu.async_copy(propS.at[pl.ds(o, ZR)],
                             prop_out.at[c, pl.ds(o, ZR)], sem).wait()
            pltpu.async_copy(denS.at[pl.ds(o, ZR)],
                             den_out.at[c, pl.ds(o, ZR)], sem2).wait()


_sc_heavy = functools.partial(
    pl.kernel,
    out_type=(
        jax.ShapeDtypeStruct((NC, N, CH), jnp.float32),
        jax.ShapeDtypeStruct((NC, N, 16), jnp.float32),
    ),
    mesh=plsc.VectorSubcoreMesh(core_axis_name="c", subcore_axis_name="s"),
    compiler_params=pltpu.CompilerParams(
        needs_layout_passes=False, use_tc_tiling_on_sc=False),
    scratch_types=[
        pltpu.VMEM((SB, K), jnp.int32),
        pltpu.VMEM((SB, K), jnp.int32),
        pltpu.VMEM((SB, K), jnp.float32),
        pltpu.VMEM((SB, K), jnp.int32),
        pltpu.VMEM((K, CH), jnp.float32),
        pltpu.VMEM((K, CH), jnp.float32),
        pltpu.VMEM((K, 16), jnp.float32),
        pltpu.VMEM((ZR, CH), jnp.float32),
        pltpu.VMEM((ZR, 16), jnp.float32),
        pltpu.MemorySpace.VMEM_SHARED((N, CH), jnp.float32),
        pltpu.MemorySpace.VMEM_SHARED((N, 16), jnp.float32),
        pltpu.SemaphoreType.DMA,
        pltpu.SemaphoreType.DMA,
        pltpu.SemaphoreType.DMA,
        pltpu.SemaphoreType.DMA,
    ],
)(_sc_heavy_body)


@jax.jit
def kernel(x, edge_index, gate_w, gate_b):
    pad = NW * EPP - E
    row = jnp.pad(edge_index[0].astype(jnp.int32).reshape(NW, EPT),
                  ((0, 0), (0, EPP - EPT))).reshape(NW, NBP, K)
    col = jnp.pad(edge_index[1].astype(jnp.int32).reshape(NW, EPT),
                  ((0, 0), (0, EPP - EPT))).reshape(NW, NBP, K)
    del pad

    wcat = jnp.concatenate(
        [gate_w[0, :C, None], gate_w[0, C:, None]], axis=1)
    bias = jnp.stack([jnp.zeros((), jnp.float32), gate_b[0]])[None, :]
    BN = 2000
    s2 = pl.pallas_call(
        _gate_kernel,
        out_shape=jax.ShapeDtypeStruct((N, 2), jnp.float32),
        grid=(N // BN,),
        in_specs=[
            pl.BlockSpec((BN, C), lambda i: (i, 0)),
            pl.BlockSpec((C, 2), lambda i: (0, 0)),
            pl.BlockSpec((1, 2), lambda i: (0, 0)),
        ],
        out_specs=pl.BlockSpec((BN, 2), lambda i: (i, 0)),
    )(x, wcat, bias)
    s_row = s2[:, 0]
    s_col = s2[:, 1]

    ex = _sc_ex(row, col, s_row, s_col)

    row16 = row.reshape(NS, NB2, K)
    col16 = col.reshape(NS, NB2, K)
    ex16 = ex.reshape(NS, NB2, K)
    xf = x.reshape(2 * N, CH)
    prop, den = _sc_heavy(row16, col16, ex16, xf)

    R = 1000
    out = pl.pallas_call(
        _combine_kernel,
        out_shape=jax.ShapeDtypeStruct((N, C), jnp.float32),
        grid=(N // R,),
        in_specs=[
            pl.BlockSpec((NC, R, C // NC), lambda i: (0, i, 0)),
            pl.BlockSpec((NC, R, 16), lambda i: (0, i, 0)),
            pl.BlockSpec((R, C), lambda i: (i, 0)),
        ],
        out_specs=pl.BlockSpec((R, C), lambda i: (i, 0)),
    )(prop, den, x)
    return out
